# Initial kernel scaffold; baseline (speedup 1.0000x reference)
import jax
import jax.numpy as jnp
from jax.experimental import pallas as pl


def kernel(indices, table1, table2):
    raise NotImplementedError("write your pallas kernel here")


# speedup vs baseline: 1.1646x; 1.1646x over previous
import jax
import jax.numpy as jnp
from jax import lax
from jax.experimental import pallas as pl
from jax.experimental.pallas import tpu as pltpu
from jax.experimental.pallas import tpu_sc as plsc

NC = 2
NS = 16
NW = NC * NS
L = 16

PAD_ROWS = 8
CHUNK = 128


def _build(B, D, V1):
    b_per_w = B // NW
    n_chunks = b_per_w // CHUNK
    n_vregs = b_per_w // L
    mesh = plsc.VectorSubcoreMesh(
        core_axis_name="c", subcore_axis_name="s",
        num_cores=NC, num_subcores=NS)

    def body(idx_hbm, t1_hbm, t2_hbm, out_hbm,
             idx_v, i1_v, i2_v, p1_v, p2_v, rows1, rows2,
             sg1, sg2, ss):
        wid = lax.axis_index("s") * NC + lax.axis_index("c")
        base = wid * b_per_w
        dump = B + lax.rem(wid, PAD_ROWS)

        pltpu.sync_copy(idx_hbm.at[pl.ds(base, b_per_w)], idx_v)

        iota = lax.iota(jnp.int32, L)
        for i in range(n_vregs):
            v = idx_v[pl.ds(i * L, L)]
            m = v < V1
            row = base + i * L + iota
            j, k = divmod(i * L, CHUNK)
            sl = pl.ds(k, L)
            i1_v[j, sl] = jnp.where(m, v, 0)
            i2_v[j, sl] = jnp.where(m, 0, v - V1)
            p1_v[j, sl] = jnp.where(m, row, dump)
            p2_v[j, sl] = jnp.where(m, dump, row)

        g1 = [pltpu.async_copy(t1_hbm.at[i1_v.at[j]],
                               rows1.at[pl.ds(j * CHUNK, CHUNK)], sg1)
              for j in range(n_chunks)]
        g2 = [pltpu.async_copy(t2_hbm.at[i2_v.at[j]],
                               rows2.at[pl.ds(j * CHUNK, CHUNK)], sg2)
              for j in range(n_chunks)]
        sc = []
        for j in range(n_chunks):
            g1[j].wait()
            sc.append(pltpu.async_copy(rows1.at[pl.ds(j * CHUNK, CHUNK)],
                                       out_hbm.at[p1_v.at[j]], ss))
        for j in range(n_chunks):
            g2[j].wait()
            sc.append(pltpu.async_copy(rows2.at[pl.ds(j * CHUNK, CHUNK)],
                                       out_hbm.at[p2_v.at[j]], ss))
        for c in sc:
            c.wait()

    return pl.kernel(
        body,
        out_type=jax.ShapeDtypeStruct((B + PAD_ROWS, D), jnp.float32),
        mesh=mesh,
        compiler_params=pltpu.CompilerParams(use_tc_tiling_on_sc=False),
        scratch_types=[
            pltpu.VMEM((b_per_w,), jnp.int32),
            pltpu.VMEM((n_chunks, CHUNK), jnp.int32),
            pltpu.VMEM((n_chunks, CHUNK), jnp.int32),
            pltpu.VMEM((n_chunks, CHUNK), jnp.int32),
            pltpu.VMEM((n_chunks, CHUNK), jnp.int32),
            pltpu.VMEM((b_per_w, D), jnp.float32),
            pltpu.VMEM((b_per_w, D), jnp.float32),
            pltpu.SemaphoreType.DMA,
            pltpu.SemaphoreType.DMA,
            pltpu.SemaphoreType.DMA,
        ],
    )


def kernel(indices, table1, table2):
    B = indices.shape[0]
    V1, D = table1.shape
    padded = _build(B, D, V1)(indices.astype(jnp.int32), table1, table2)
    return padded[:B]
